# Initial kernel scaffold; baseline (speedup 1.0000x reference)
import jax
import jax.numpy as jnp
from jax.experimental import pallas as pl


def kernel(object_ids, table):
    raise NotImplementedError("write your pallas kernel here")


# speedup vs baseline: 6.1306x; 6.1306x over previous
import functools

import jax
import jax.numpy as jnp
from jax import lax
from jax.experimental import pallas as pl
from jax.experimental.pallas import tpu as pltpu
from jax.experimental.pallas import tpu_sc as plsc

NC = 2
NS = 16
NW = NC * NS
D = 32
IW = 128
K = 8
CHUNK = K * IW


@functools.lru_cache(maxsize=None)
def _make(B):
    assert B % (NW * CHUNK) == 0
    rows_per_w = (B // NW) // IW
    steps = rows_per_w // K
    mesh = plsc.VectorSubcoreMesh(
        core_axis_name="c", subcore_axis_name="s",
        num_cores=NC, num_subcores=NS)

    @functools.partial(
        pl.kernel,
        out_type=jax.ShapeDtypeStruct((B, D), jnp.float32),
        mesh=mesh,
        scratch_types=[
            pltpu.VMEM((K, IW), jnp.int32),
            pltpu.VMEM((CHUNK, D), jnp.float32),
            pltpu.SemaphoreType.DMA,
        ],
        compiler_params=pltpu.CompilerParams(use_tc_tiling_on_sc=False),
    )
    def k(idx_hbm, table_hbm, out_hbm, idx_v, rows_v, sem):
        wid = lax.axis_index("s") * NC + lax.axis_index("c")
        row_base = wid * rows_per_w

        def step(c, carry):
            r0 = row_base + c * K
            pltpu.sync_copy(idx_hbm.at[pl.ds(r0, K)], idx_v)
            cps = [
                pltpu.async_copy(table_hbm.at[idx_v.at[j]],
                                 rows_v.at[pl.ds(j * IW, IW)], sem)
                for j in range(K)
            ]
            for cp in cps:
                cp.wait()
            pltpu.sync_copy(rows_v, out_hbm.at[pl.ds(r0 * IW, CHUNK)])
            return carry

        lax.fori_loop(0, steps, step, 0)

    return k


def kernel(object_ids, table):
    S, T = object_ids.shape
    B = S * T
    idx = object_ids.reshape(B // IW, IW).astype(jnp.int32)
    out = _make(B)(idx, table)
    return out.reshape(S, T, D)
